# Initial kernel scaffold; baseline (speedup 1.0000x reference)
import jax
import jax.numpy as jnp
from jax.experimental import pallas as pl


def kernel(x, edge_index, y, W1, b1, W2, b2):
    raise NotImplementedError("write your pallas kernel here")


# speedup vs baseline: 19.2285x; 19.2285x over previous
import functools

import jax
import jax.numpy as jnp
from jax import lax
from jax.experimental import pallas as pl
from jax.experimental.pallas import tpu as pltpu
from jax.experimental.pallas import tpu_sc as plsc

N = 10000
E = 320000
D = 128
H = 256
C = 40
CP = 48

NC, NS, LANES = 2, 16, 16
NW = NC * NS
NPAD = 10240
EPAD = 327680
CHUNK = 128
CHUNKS_PER_W = EPAD // (NW * CHUNK)
ROWS_PER_T = NPAD // NS

_mesh = plsc.VectorSubcoreMesh(core_axis_name="c", subcore_axis_name="s")


@functools.partial(
    pl.kernel,
    out_type=jax.ShapeDtypeStruct((2 * NPAD,), jnp.float32),
    mesh=_mesh,
    scratch_types=[
        pltpu.VMEM((CHUNK,), jnp.int32),
        pltpu.VMEM((CHUNK,), jnp.float32),
        pltpu.VMEM((ROWS_PER_T,), jnp.float32),
        pltpu.VMEM_SHARED((NPAD,), jnp.float32),
    ],
)
def _deg_kernel(dst_hbm, zeros1_hbm, out_hbm, idx_d, ones_v, stage, deg_sh):
    cid = lax.axis_index("c")
    sid = lax.axis_index("s")
    wid = sid * NC + cid
    pltpu.sync_copy(zeros1_hbm.at[pl.ds(sid * ROWS_PER_T, ROWS_PER_T)], stage)
    pltpu.sync_copy(stage, deg_sh.at[pl.ds(sid * ROWS_PER_T, ROWS_PER_T)])
    for k in range(CHUNK // LANES):
        ones_v[pl.ds(k * LANES, LANES)] = jnp.ones((LANES,), jnp.float32)
    plsc.subcore_barrier()

    def step(i, carry):
        ebase = (wid * CHUNKS_PER_W + i) * CHUNK
        pltpu.sync_copy(dst_hbm.at[pl.ds(ebase, CHUNK)], idx_d)
        pltpu.sync_copy(ones_v, deg_sh.at[idx_d], add=True)
        return carry

    lax.fori_loop(0, CHUNKS_PER_W, step, 0)
    plsc.subcore_barrier()
    pltpu.sync_copy(deg_sh.at[pl.ds(sid * ROWS_PER_T, ROWS_PER_T)], stage)
    pltpu.sync_copy(
        stage, out_hbm.at[pl.ds(cid * NPAD + sid * ROWS_PER_T, ROWS_PER_T)])


def _make_prop(width):
    @functools.partial(
        pl.kernel,
        out_type=jax.ShapeDtypeStruct((2 * NPAD, width), jnp.float32),
        mesh=_mesh,
        compiler_params=pltpu.CompilerParams(use_tc_tiling_on_sc=(width % 128 == 0)),
        scratch_types=[
            pltpu.VMEM((CHUNK,), jnp.int32),
            pltpu.VMEM((CHUNK,), jnp.int32),
            pltpu.VMEM((CHUNK, width), jnp.float32),
            pltpu.VMEM_SHARED((NPAD, width), jnp.float32),
            pltpu.SemaphoreType.DMA,
        ],
    )
    def _prop(xs_hbm, srcp_hbm, dstp_hbm, zeros_hbm, out_hbm,
              idx_s, idx_d, rows, acc_sh, sem):
        cid = lax.axis_index("c")
        sid = lax.axis_index("s")
        wid = sid * NC + cid
        for p in range(ROWS_PER_T // CHUNK):
            r0 = sid * ROWS_PER_T + p * CHUNK
            pltpu.sync_copy(zeros_hbm.at[pl.ds(r0, CHUNK)],
                            acc_sh.at[pl.ds(r0, CHUNK)])
        plsc.subcore_barrier()

        def step(i, carry):
            ebase = (wid * CHUNKS_PER_W + i) * CHUNK
            pltpu.sync_copy(srcp_hbm.at[pl.ds(ebase, CHUNK)], idx_s)
            pltpu.sync_copy(dstp_hbm.at[pl.ds(ebase, CHUNK)], idx_d)
            pltpu.async_copy(xs_hbm.at[idx_s], rows, sem).wait()
            pltpu.sync_copy(rows, acc_sh.at[idx_d], add=True)
            return carry

        lax.fori_loop(0, CHUNKS_PER_W, step, 0)
        plsc.subcore_barrier()
        for p in range(ROWS_PER_T // CHUNK):
            r0 = sid * ROWS_PER_T + p * CHUNK
            pltpu.sync_copy(acc_sh.at[pl.ds(r0, CHUNK)],
                            out_hbm.at[pl.ds(cid * NPAD + r0, CHUNK)])

    return _prop


_prop128 = _make_prop(D)
_prop48 = _make_prop(CP)


def _prep_body(deg0, deg1, x_ref, dinv_out, xs_out):
    dg = deg0[...] + deg1[...] + 1.0
    di = lax.rsqrt(dg)
    dinv_out[...] = di
    xs_out[...] = x_ref[...] * di


def _mid_body(a0, a1, xs, dinv, W1r, b1r, W2r, zs_out):
    g = (a0[...] + a1[...] + xs[...]) * dinv[...]
    h = jnp.dot(g, W1r[...], preferred_element_type=jnp.float32) + b1r[...]
    h = jnp.maximum(h, 0.0)
    z = jnp.dot(h, W2r[...], preferred_element_type=jnp.float32)
    zs_out[...] = z * dinv[...]


def _loss_body(a0, a1, zs, dinv, b2r, y_ref, out_ref):
    i = pl.program_id(0)
    rows = a0.shape[0]
    o = (a0[...] + a1[...] + zs[...]) * dinv[...] + b2r[...]
    col = lax.broadcasted_iota(jnp.int32, (rows, CP), 1)
    o = jnp.where(col < C, o, -1e30)
    m = jnp.max(o, axis=1, keepdims=True)
    lse = jnp.log(jnp.sum(jnp.exp(o - m), axis=1, keepdims=True)) + m
    picked = jnp.sum(jnp.where(col == y_ref[...], o - lse, 0.0), axis=1)
    part = jnp.sum(picked) * (-1.0 / N)

    @pl.when(i == 0)
    def _():
        out_ref[...] = jnp.zeros_like(out_ref)

    out_ref[...] += jnp.full((1, 1), 1.0, jnp.float32) * part


def kernel(x, edge_index, y, W1, b1, W2, b2):
    f32 = jnp.float32
    src = edge_index[0]
    dst = edge_index[1]
    pad = (jnp.arange(EPAD - E, dtype=jnp.int32) % (NPAD - N)) + N
    srcp = jnp.concatenate([src, pad])
    dstp = jnp.concatenate([dst, pad])
    xpad = jnp.zeros((NPAD, D), f32).at[:N].set(x)
    zeros1 = jnp.zeros((NPAD,), f32)
    zeros2 = jnp.zeros((NPAD, D), f32)
    zeros48 = jnp.zeros((NPAD, CP), f32)
    W2p = jnp.pad(W2, ((0, 0), (0, CP - C)))
    b1r = b1.reshape(1, H)
    b2r = jnp.pad(b2, (0, CP - C)).reshape(1, CP)

    degp = _deg_kernel(dstp, zeros1)
    deg0 = degp[:NPAD].reshape(NPAD, 1)
    deg1 = degp[NPAD:].reshape(NPAD, 1)

    R = 1024
    G = NPAD // R
    dinv, xs = pl.pallas_call(
        _prep_body,
        grid=(G,),
        in_specs=[
            pl.BlockSpec((R, 1), lambda i: (i, 0)),
            pl.BlockSpec((R, 1), lambda i: (i, 0)),
            pl.BlockSpec((R, D), lambda i: (i, 0)),
        ],
        out_specs=[
            pl.BlockSpec((R, 1), lambda i: (i, 0)),
            pl.BlockSpec((R, D), lambda i: (i, 0)),
        ],
        out_shape=[
            jax.ShapeDtypeStruct((NPAD, 1), f32),
            jax.ShapeDtypeStruct((NPAD, D), f32),
        ],
    )(deg0, deg1, xpad)

    acc1 = _prop128(xs, srcp, dstp, zeros2)

    zs = pl.pallas_call(
        _mid_body,
        grid=(G,),
        in_specs=[
            pl.BlockSpec((R, D), lambda i: (i, 0)),
            pl.BlockSpec((R, D), lambda i: (i + G, 0)),
            pl.BlockSpec((R, D), lambda i: (i, 0)),
            pl.BlockSpec((R, 1), lambda i: (i, 0)),
            pl.BlockSpec((D, H), lambda i: (0, 0)),
            pl.BlockSpec((1, H), lambda i: (0, 0)),
            pl.BlockSpec((H, CP), lambda i: (0, 0)),
        ],
        out_specs=pl.BlockSpec((R, CP), lambda i: (i, 0)),
        out_shape=jax.ShapeDtypeStruct((NPAD, CP), f32),
    )(acc1, acc1, xs, dinv, W1, b1r, W2p)

    acc2 = _prop48(zs, srcp, dstp, zeros48)

    RL = 1000
    GL = N // RL
    a20 = acc2[:N]
    a21 = acc2[NPAD:NPAD + N]
    out = pl.pallas_call(
        _loss_body,
        grid=(GL,),
        in_specs=[
            pl.BlockSpec((RL, CP), lambda i: (i, 0)),
            pl.BlockSpec((RL, CP), lambda i: (i, 0)),
            pl.BlockSpec((RL, CP), lambda i: (i, 0)),
            pl.BlockSpec((RL, 1), lambda i: (i, 0)),
            pl.BlockSpec((1, CP), lambda i: (0, 0)),
            pl.BlockSpec((RL, 1), lambda i: (i, 0)),
        ],
        out_specs=pl.BlockSpec((1, 1), lambda i: (0, 0)),
        out_shape=jax.ShapeDtypeStruct((1, 1), f32),
    )(a20, a21, zs[:N], dinv[:N], b2r, y)

    return out[0, 0]
